# baseline (speedup 1.0000x reference)
import functools

import jax
import jax.numpy as jnp
from jax import lax
from jax.experimental import pallas as pl
from jax.experimental.pallas import tpu as pltpu
from jax.experimental.pallas import tpu_sc as plsc

N_EMB = 1_000_000
D_EMB = 64
BATCH = 16384

_NC = 2
_NS = 16
_NW = _NC * _NS
_CH = 128
_NCH = BATCH // (_NW * _CH)
_NCHUNKS = BATCH // _CH
_NTILES = BATCH // (8 * _CH)

_mesh = plsc.VectorSubcoreMesh(core_axis_name="c", subcore_axis_name="s")


@functools.partial(
    pl.kernel,
    mesh=_mesh,
    out_type=jax.ShapeDtypeStruct((_NCHUNKS, _CH, 2 * D_EMB), jnp.float32),
    scratch_types=[
        pltpu.VMEM((8, _CH), jnp.int32),
        pltpu.VMEM((_CH, 2 * D_EMB), jnp.float32),
        pltpu.VMEM((_CH, 2 * D_EMB), jnp.float32),
        pltpu.VMEM((_CH, 2 * D_EMB), jnp.float32),
        pltpu.VMEM((_CH, 2 * D_EMB), jnp.float32),
        pltpu.SemaphoreType.DMA,
        pltpu.SemaphoreType.DMA,
        pltpu.SemaphoreType.DMA,
        pltpu.SemaphoreType.DMA,
        pltpu.SemaphoreType.DMA,
    ],
)
def _pair_gather(idx_hbm, table_hbm, out_hbm,
                 idx_v, b0, b1, b2, b3, sg0, sg1, sg2, sg3, so):
    wid = lax.axis_index("s") * _NC + lax.axis_index("c")
    pltpu.sync_copy(idx_hbm.at[wid // 2], idx_v)
    r0 = (wid % 2) * _NCH
    bufs = (b0, b1, b2, b3)
    sems = (sg0, sg1, sg2, sg3)
    gathers = [
        pltpu.async_copy(table_hbm.at[idx_v.at[r0 + j]], bufs[j], sems[j])
        for j in range(_NCH)
    ]
    writes = []
    for j in range(_NCH):
        gathers[j].wait()
        writes.append(pltpu.async_copy(bufs[j], out_hbm.at[wid * _NCH + j], so))
    for w in writes:
        w.wait()


def kernel(x, emb):
    xi = x.astype(jnp.int32)
    xp = (xi >> 1).reshape(_NTILES, 8, _CH)
    table = emb.reshape(N_EMB // 2, 2 * D_EMB)
    pairs = _pair_gather(xp, table).reshape(BATCH, 2 * D_EMB)
    odd = (xi & 1).astype(jnp.bool_)[:, None]
    return jnp.where(odd, pairs[:, D_EMB:], pairs[:, :D_EMB])
